# baseline (speedup 1.0000x reference)
import functools

import jax
import jax.numpy as jnp
from jax import lax
from jax.experimental import pallas as pl
from jax.experimental.pallas import tpu as pltpu
from jax.experimental.pallas import tpu_sc as plsc

_B = 16384
_NF = 26
_VOCAB = 100000
_K = 16
_TOT = _NF * _VOCAB
_NE = _B * _NF
_NW = 16
_EPW = _NE // _NW
_CH = 4096
_NCHUNK = (_TOT + _CH - 1) // _CH
_NCHUNK_PAD = 640
_LAST_W = 3072
_LAST_FULL = _TOT - (_NCHUNK - 1) * _CH
_TAIL0 = _NCHUNK - 1
_CSR_CAP = _NE + _NCHUNK_PAD * 16
_SUB = 1024
_NSUB = _EPW // _SUB
_SLAB = 64
_RPW = _B // _NW
_AR = _B // 8


def _scalar(vec):
    return lax.reduce_max(vec, (0,))


def _bin_body(chunk_hbm, pk_hbm, csr_s, meta_hbm,
              cntT_s,
              slab_v, pkslab_v, pos_v, cnts_v, tot_v, base_v, rb_v, cntT_v,
              sem):
    sc = lax.axis_index("c")
    w = lax.axis_index("s")
    i16 = lax.iota(jnp.int32, 16)
    z16i = jnp.zeros((16,), jnp.int32)
    z16f = jnp.zeros((16,), jnp.float32)

    for cg in range(_NCHUNK_PAD // 16):
        cnts_v[pl.ds(cg * 16, 16)] = z16i

    for t in range(_NSUB):
        src = pl.multiple_of(w * _EPW + t * _SUB, _SUB)
        pltpu.sync_copy(chunk_hbm.at[pl.ds(src, _SUB)], slab_v)

        def cbody(i, _):
            ch = slab_v[pl.ds(i * 16, 16)]
            occ, last = plsc.scan_count(ch)
            plsc.addupdate_scatter(cnts_v, [ch], occ, mask=last)
            return 0
        lax.fori_loop(0, _SUB // 16, cbody, 0)

    for cg in range(_NCHUNK_PAD // 16):
        pos_v[pl.ds(cg * 16, 16)] = (cg * 16 + i16) * 16 + w
    pltpu.sync_copy(cnts_v, cntT_s.at[pos_v.at[pl.ds(0, _NCHUNK_PAD)]])
    plsc.subcore_barrier()

    pltpu.sync_copy(cntT_s, cntT_v)
    for cg in range(_NCHUNK_PAD // 16):
        cvec = (cg * 16 + i16) * 16
        tot = z16i
        off = z16i
        for wp in range(_NW):
            g = plsc.load_gather(cntT_v, [cvec + wp])
            tot = tot + g
            off = off + jnp.where(wp < w, g, z16i)
        tot_v[pl.ds(cg * 16, 16)] = tot
        rb_v[pl.ds(cg * 16, 16)] = off

    run = z16i
    for cg in range(_NCHUNK_PAD // 16):
        v = tot_v[pl.ds(cg * 16, 16)]
        vp = (v + 15) & ~15
        c = plsc.cumsum(vp)
        base_v[pl.ds(cg * 16, 16)] = c - vp + run
        run = run + _scalar(c)

    for cg in range(_NCHUNK_PAD // 16):
        rb_v[pl.ds(cg * 16, 16)] = (rb_v[pl.ds(cg * 16, 16)]
                                    + base_v[pl.ds(cg * 16, 16)])

    for t in range(_NSUB):
        src = pl.multiple_of(w * _EPW + t * _SUB, _SUB)
        pltpu.sync_copy(chunk_hbm.at[pl.ds(src, _SUB)], slab_v)
        pltpu.sync_copy(pk_hbm.at[pl.ds(src, _SUB)], pkslab_v)

        def sbody(i, _):
            ch = slab_v[pl.ds(i * 16, 16)]
            occ, last = plsc.scan_count(ch)
            p = plsc.load_gather(rb_v, [ch]) + occ - 1
            plsc.addupdate_scatter(rb_v, [ch], occ, mask=last)
            pos_v[pl.ds(i * 16, 16)] = p + sc * _CSR_CAP
            return 0
        lax.fori_loop(0, _SUB // 16, sbody, 0)
        pltpu.sync_copy(pkslab_v, csr_s.at[pos_v])

    @pl.when((sc == 0) & (w == 0))
    def _():
        pltpu.sync_copy(tot_v, meta_hbm.at[pl.ds(0, _NCHUNK_PAD)])
        pltpu.sync_copy(base_v, meta_hbm.at[pl.ds(_NCHUNK_PAD, _NCHUNK_PAD)])


def _fm_body(csr_s, meta_hbm, qtt_hbm, lin_hbm, tail_hbm,
             outq_hbm, outl_hbm,
             sacc_s, ssacc_s,
             win_v, linw_v, bb_v, csrslab_v,
             sval_v, ssval_v, tot_v, base_v,
             sbuf_v, ssbuf_v, tail_v, outq_v, outl_v,
             sem):
    sc = lax.axis_index("c")
    w = lax.axis_index("s")
    i16 = lax.iota(jnp.int32, 16)
    z16f = jnp.zeros((16,), jnp.float32)

    pltpu.sync_copy(meta_hbm.at[pl.ds(0, _NCHUNK_PAD)], tot_v)
    pltpu.sync_copy(meta_hbm.at[pl.ds(_NCHUNK_PAD, _NCHUNK_PAD)], base_v)

    def zrow(r, _):
        for cb in range(8):
            sval_v[r, pl.ds(cb * 16, 16)] = z16f
            ssval_v[r, pl.ds(cb * 16, 16)] = z16f
        return 0
    lax.fori_loop(0, _SLAB, zrow, 0)

    for h2 in range(2):
        arow = pl.multiple_of(w * (_AR // _NW) + h2 * _SLAB, _SLAB)
        pltpu.sync_copy(sval_v, sacc_s.at[pl.ds(arow, _SLAB), :])
        pltpu.sync_copy(ssval_v, ssacc_s.at[pl.ds(arow, _SLAB), :])
    plsc.subcore_barrier()

    pltpu.sync_copy(tail_hbm, tail_v)
    scrow = pl.multiple_of(8 * sc, 8)

    def do_slabs(n, seg, width, is_last):
        def slab_body(t, _):
            s0 = pl.multiple_of(seg + sc * _CSR_CAP + t * _SLAB, 16)
            pltpu.sync_copy(csr_s.at[pl.ds(s0, _SLAB)], csrslab_v)

            def grp(i, fill):
                pk = csrslab_v[pl.ds(i * 16, 16)]
                valid = (t * _SLAB + i * 16 + i16) < n
                b = (pk >> 13) & 16383
                lx = jnp.minimum(pk & 8191, width - 1)
                erow = i * 16 + i16
                blk = (b & 7) * 16
                if not fill:
                    for kk in range(8):
                        kv = blk + kk
                        plsc.store_scatter(sval_v, [erow, kv], z16f)
                        plsc.store_scatter(ssval_v, [erow, kv], z16f)
                    plsc.store_scatter(sval_v, [erow, blk + 8], z16f)
                    return
                for kk in range(8):
                    kv = jnp.full((16,), kk, jnp.int32)
                    v = plsc.load_gather(win_v, [kv, lx])
                    if is_last:
                        tl = (jnp.maximum(lx - _LAST_W, 0)
                              + (8 * sc + kk) * 64)
                        vt = plsc.load_gather(tail_v, [tl])
                        v = jnp.where(lx < _LAST_W, v, vt)
                    vm = jnp.where(valid, v, z16f)
                    plsc.store_scatter(sval_v, [erow, blk + kk], vm)
                    plsc.store_scatter(ssval_v, [erow, blk + kk], vm * vm)
                lv = plsc.load_gather(linw_v, [lx])
                lvm = jnp.where(valid & (sc == 0), lv, z16f)
                plsc.store_scatter(sval_v, [erow, blk + 8], lvm)
                bb_v[pl.ds(i * 16, 16)] = b >> 3

            def grp_fill(i, _):
                grp(i, True)
                return 0

            def grp_clean(i, _):
                grp(i, False)
                return 0

            lax.fori_loop(0, _SLAB // 16, grp_fill, 0)
            pltpu.sync_copy(sval_v, sacc_s.at[bb_v], add=True)
            pltpu.sync_copy(ssval_v, ssacc_s.at[bb_v], add=True)
            lax.fori_loop(0, _SLAB // 16, grp_clean, 0)
            return 0

        nslab = (n + _SLAB - 1) >> 6
        lax.fori_loop(0, nslab, slab_body, 0)

    def full_chunk(cid):
        col0 = pl.multiple_of(cid * _CH, _CH)
        pltpu.sync_copy(qtt_hbm.at[pl.ds(scrow, 8), pl.ds(col0, _CH)], win_v)
        pltpu.sync_copy(lin_hbm.at[pl.ds(col0, _CH)], linw_v)
        cidv = jnp.full((16,), cid, jnp.int32)
        n = _scalar(plsc.load_gather(tot_v, [cidv]))
        seg = _scalar(plsc.load_gather(base_v, [cidv]))
        do_slabs(n, seg, _CH, False)

    def jbody(j, _):
        cid = w + 16 * j

        @pl.when(cid < _TAIL0)
        def _():
            full_chunk(cid)

        @pl.when(cid == _TAIL0)
        def _():
            col0 = _TAIL0 * _CH
            pltpu.sync_copy(
                qtt_hbm.at[pl.ds(scrow, 8), pl.ds(col0, _LAST_W)],
                win_v.at[pl.ds(0, 8), pl.ds(0, _LAST_W)])
            pltpu.sync_copy(lin_hbm.at[pl.ds(col0, _LAST_FULL)],
                            linw_v.at[pl.ds(0, _LAST_FULL)])
            cidv = jnp.full((16,), _TAIL0, jnp.int32)
            n = _scalar(plsc.load_gather(tot_v, [cidv]))
            seg = _scalar(plsc.load_gather(base_v, [cidv]))
            do_slabs(n, seg, _LAST_FULL, True)
        return 0

    lax.fori_loop(0, (_NCHUNK + _NW - 1) // _NW, jbody, 0)

    plsc.subcore_barrier()

    for h in range(2):
        rbase = pl.multiple_of(w * (_AR // _NW) + h * 64, 64)
        pltpu.sync_copy(sacc_s.at[pl.ds(rbase, 64), :], sbuf_v)
        pltpu.sync_copy(ssacc_s.at[pl.ds(rbase, 64), :], ssbuf_v)

        def fgrp(i, _):
            rl = i * 16 + i16
            row = rl >> 3
            col0 = (rl & 7) * 16
            q = z16f
            for kk in range(8):
                sv = plsc.load_gather(sbuf_v, [row, col0 + kk])
                ssv = plsc.load_gather(ssbuf_v, [row, col0 + kk])
                q = q + (sv * sv - ssv)
            outq_v[pl.ds(h * 512 + i * 16, 16)] = 0.5 * q
            lv = plsc.load_gather(sbuf_v, [row, col0 + 8])
            outl_v[pl.ds(h * 512 + i * 16, 16)] = lv
            return 0
        lax.fori_loop(0, 512 // 16, fgrp, 0)

    oq = pl.multiple_of(sc * _B + w * _RPW, _RPW)
    pltpu.sync_copy(outq_v, outq_hbm.at[pl.ds(oq, _RPW)])

    @pl.when(sc == 0)
    def _():
        ol = pl.multiple_of(w * _RPW, _RPW)
        pltpu.sync_copy(outl_v, outl_hbm.at[pl.ds(ol, _RPW)])


@jax.jit
def _fm_sc(chunkR, pkR, qtt, linf, tailq):
    mesh = plsc.VectorSubcoreMesh(core_axis_name="c", subcore_axis_name="s")
    binrun = functools.partial(
        pl.kernel,
        out_type=[
            jax.ShapeDtypeStruct((2 * _CSR_CAP,), jnp.int32),
            jax.ShapeDtypeStruct((2 * _NCHUNK_PAD,), jnp.int32),
        ],
        mesh=mesh,
        compiler_params=pltpu.CompilerParams(needs_layout_passes=False),
        scratch_types=[
            pltpu.VMEM_SHARED((_NCHUNK_PAD * _NW,), jnp.int32),
            pltpu.VMEM((_SUB,), jnp.int32),
            pltpu.VMEM((_SUB,), jnp.int32),
            pltpu.VMEM((_SUB,), jnp.int32),
            pltpu.VMEM((_NCHUNK_PAD,), jnp.int32),
            pltpu.VMEM((_NCHUNK_PAD,), jnp.int32),
            pltpu.VMEM((_NCHUNK_PAD,), jnp.int32),
            pltpu.VMEM((_NCHUNK_PAD,), jnp.int32),
            pltpu.VMEM((_NCHUNK_PAD * _NW,), jnp.int32),
            pltpu.SemaphoreType.DMA,
        ],
    )(_bin_body)
    csr, meta = binrun(chunkR, pkR)

    mainrun = functools.partial(
        pl.kernel,
        out_type=[
            jax.ShapeDtypeStruct((2 * _B,), jnp.float32),
            jax.ShapeDtypeStruct((_B,), jnp.float32),
        ],
        mesh=mesh,
        compiler_params=pltpu.CompilerParams(needs_layout_passes=False),
        scratch_types=[
            pltpu.VMEM_SHARED((_AR, 128), jnp.float32),
            pltpu.VMEM_SHARED((_AR, 128), jnp.float32),
            pltpu.VMEM((8, _CH), jnp.float32),
            pltpu.VMEM((_CH,), jnp.float32),
            pltpu.VMEM((_SLAB,), jnp.int32),
            pltpu.VMEM((_SLAB,), jnp.int32),
            pltpu.VMEM((_SLAB, 128), jnp.float32),
            pltpu.VMEM((_SLAB, 128), jnp.float32),
            pltpu.VMEM((_NCHUNK_PAD,), jnp.int32),
            pltpu.VMEM((_NCHUNK_PAD,), jnp.int32),
            pltpu.VMEM((64, 128), jnp.float32),
            pltpu.VMEM((64, 128), jnp.float32),
            pltpu.VMEM((1024,), jnp.float32),
            pltpu.VMEM((_RPW,), jnp.float32),
            pltpu.VMEM((_RPW,), jnp.float32),
            pltpu.SemaphoreType.DMA,
        ],
    )(_fm_body)
    outq2, outl = mainrun(csr, meta, qtt, linf, tailq)
    return outq2, outl


def kernel(input, quad_table, lin_table, global_bias):
    inp = input.astype(jnp.int32)
    offsets = (jnp.arange(_NF, dtype=jnp.int32) * _VOCAB)
    gcol = inp + offsets[None, :]
    chunk = gcol // _CH
    lx = gcol - chunk * _CH
    b = jnp.arange(_B, dtype=jnp.int32)[:, None]
    pk = (b << 13) | lx
    chunkR = chunk.reshape(-1)
    pkR = pk.reshape(-1)
    qtt = quad_table.T
    linf = lin_table.reshape(-1)
    tailq = quad_table[_TAIL0 * _CH + _LAST_W:, :].T.reshape(-1)
    outq2, linear = _fm_sc(chunkR, pkR, qtt, linf, tailq)
    quadratic = outq2[:_B] + outq2[_B:]
    bias = jnp.broadcast_to(global_bias, (_B,))
    return (quadratic, linear, bias)


# speedup vs baseline: 1.5258x; 1.5258x over previous
import functools

import jax
import jax.numpy as jnp
from jax import lax
from jax.experimental import pallas as pl
from jax.experimental.pallas import tpu as pltpu
from jax.experimental.pallas import tpu_sc as plsc

_B = 16384
_N_FIELDS = 26
_VOCAB = 100000
_K = 16
_TOTAL = _N_FIELDS * _VOCAB

_NC = 2
_NS = 16
_NW = _NC * _NS
_ROWS_PER_W = _B // _NW
_CHUNK_ROWS = 64
_IDX_PER_CHUNK = _CHUNK_ROWS * _N_FIELDS
_IDX_VECS = _IDX_PER_CHUNK // 128
_NCHUNKS = _ROWS_PER_W // _CHUNK_ROWS
_GROUPS = _CHUNK_ROWS // 16


def _fm_body(idx_hbm, quad_hbm, lin_hbm, outq_hbm, outl_hbm,
             idx_v, qrows_v, lrows_v, outq_v, outl_v, qsem, lsem):
    wid = lax.axis_index("s") * _NC + lax.axis_index("c")

    pltpu.sync_copy(idx_hbm.at[pl.ds(wid * (_NCHUNKS * _IDX_VECS),
                                     _NCHUNKS * _IDX_VECS)], idx_v)

    for c in range(_NCHUNKS):
        qcopies = [
            pltpu.async_copy(quad_hbm.at[idx_v.at[c * _IDX_VECS + j]],
                             qrows_v.at[pl.ds(j * 128, 128)], qsem)
            for j in range(_IDX_VECS)
        ]
        lcopies = [
            pltpu.async_copy(lin_hbm.at[idx_v.at[c * _IDX_VECS + j]],
                             lrows_v.at[pl.ds(j * 128, 128)], lsem)
            for j in range(_IDX_VECS)
        ]
        for cp in qcopies:
            cp.wait()
        for cp in lcopies:
            cp.wait()

        def group_body(g, _, c=c):
            rloc = g * 16 + lax.iota(jnp.int32, 16)
            ebase = rloc * _N_FIELDS
            zeros16 = jnp.zeros((16,), jnp.int32)

            lacc = jnp.zeros((16,), jnp.float32)
            for f in range(_N_FIELDS):
                lacc = lacc + plsc.load_gather(lrows_v, [ebase + f])

            def k_body(k, qacc):
                kvec = jnp.full((16,), k, jnp.int32)
                s = jnp.zeros((16,), jnp.float32)
                ss = jnp.zeros((16,), jnp.float32)
                for f in range(_N_FIELDS):
                    v = plsc.load_gather(qrows_v, [ebase + f, kvec])
                    s = s + v
                    ss = ss + v * v
                return qacc + (s * s - ss)

            qacc = lax.fori_loop(0, _K, k_body, jnp.zeros((16,), jnp.float32))

            off = c * _CHUNK_ROWS + g * 16
            outq_v[pl.ds(off, 16)] = 0.5 * qacc
            outl_v[pl.ds(off, 16)] = lacc
            return 0

        lax.fori_loop(0, _GROUPS, group_body, 0)

    out_base = wid * _ROWS_PER_W
    pltpu.sync_copy(outq_v, outq_hbm.at[pl.ds(out_base, _ROWS_PER_W)])
    pltpu.sync_copy(outl_v, outl_hbm.at[pl.ds(out_base, _ROWS_PER_W)])


@jax.jit
def _fm_sc(idx2d, quad_table, lin_table):
    mesh = plsc.VectorSubcoreMesh(core_axis_name="c", subcore_axis_name="s")
    run = functools.partial(
        pl.kernel,
        out_type=[
            jax.ShapeDtypeStruct((_B,), jnp.float32),
            jax.ShapeDtypeStruct((_B,), jnp.float32),
        ],
        mesh=mesh,
        compiler_params=pltpu.CompilerParams(use_tc_tiling_on_sc=False,
                                             needs_layout_passes=False),
        scratch_types=[
            pltpu.VMEM((_NCHUNKS * _IDX_VECS, 128), jnp.int32),
            pltpu.VMEM((_IDX_PER_CHUNK, _K), jnp.float32),
            pltpu.VMEM((_IDX_PER_CHUNK,), jnp.float32),
            pltpu.VMEM((_ROWS_PER_W,), jnp.float32),
            pltpu.VMEM((_ROWS_PER_W,), jnp.float32),
            pltpu.SemaphoreType.DMA,
            pltpu.SemaphoreType.DMA,
        ],
    )(_fm_body)
    return run(idx2d, quad_table, lin_table)


def kernel(input, quad_table, lin_table, global_bias):
    offsets = (jnp.arange(_N_FIELDS, dtype=jnp.int32) * _VOCAB)
    idx = input.astype(jnp.int32) + offsets[None, :]
    idx2d = idx.reshape(-1, 128)
    quadratic, linear = _fm_sc(idx2d, quad_table, lin_table.reshape(-1))
    bias = jnp.broadcast_to(global_bias, (input.shape[0],))
    return (quadratic, linear, bias)
